# Initial kernel scaffold; baseline (speedup 1.0000x reference)
import jax
import jax.numpy as jnp
from jax.experimental import pallas as pl


def kernel(features, edge_index, W1l, b1, W1r, W2l, b2, W2r):
    raise NotImplementedError("write your pallas kernel here")


# speedup vs baseline: 3.6450x; 3.6450x over previous
import functools

import jax
import jax.numpy as jnp
from jax import lax
from jax.experimental import pallas as pl
from jax.experimental.pallas import tpu as pltpu
from jax.experimental.pallas import tpu_sc as plsc

N = 10000
E = 320000
NFEAT = 128
NHID = 128
NCLASS = 64

NC = 2
NS = 16
NW = NC * NS
BATCH = 128
NB = 80
CHUNK = 8
EPAD = NW * NB * BATCH
NACC = N + 240
ROWS_PER_TILE = NACC // NS

_MESH = plsc.VectorSubcoreMesh(core_axis_name="c", subcore_axis_name="s")


def _seg_sum_cnt(table, srcs, dsts, zfeat, zcnt, ones16):
    @functools.partial(
        pl.kernel,
        out_type=[
            jax.ShapeDtypeStruct((NC, NACC, NFEAT), jnp.float32),
            jax.ShapeDtypeStruct((NC, NACC, 8), jnp.float32),
        ],
        mesh=_MESH,
        compiler_params=pltpu.CompilerParams(use_tc_tiling_on_sc=False),
        scratch_types=[
            pltpu.VMEM((CHUNK, BATCH), jnp.int32),
            pltpu.VMEM((CHUNK, BATCH), jnp.int32),
            pltpu.VMEM((BATCH, NFEAT), jnp.float32),
            pltpu.VMEM((BATCH, 8), jnp.float32),
            pltpu.VMEM_SHARED((NACC, NFEAT), jnp.float32),
            pltpu.VMEM_SHARED((NACC, 8), jnp.float32),
        ],
    )
    def k(table_hbm, src_hbm, dst_hbm, zf_hbm, zc_hbm, ones_hbm,
          osum_hbm, ocnt_hbm, src_v, dst_v, rows_v, ones_v, acc_s, acc_c):
        c = lax.axis_index("c")
        s = lax.axis_index("s")
        wid = s * NC + c
        pltpu.sync_copy(ones_hbm, ones_v)
        r0 = s * ROWS_PER_TILE
        sl = pl.ds(r0, ROWS_PER_TILE)
        pltpu.sync_copy(zf_hbm.at[sl], acc_s.at[sl])
        pltpu.sync_copy(zc_hbm.at[sl], acc_c.at[sl])
        plsc.subcore_barrier()

        def outer(ci, carry):
            pltpu.sync_copy(src_hbm.at[wid, pl.ds(ci * CHUNK, CHUNK)], src_v)
            pltpu.sync_copy(dst_hbm.at[wid, pl.ds(ci * CHUNK, CHUNK)], dst_v)

            def body(j, c2):
                pltpu.sync_copy(table_hbm.at[src_v.at[j]], rows_v)
                pltpu.sync_copy(rows_v, acc_s.at[dst_v.at[j]], add=True)
                pltpu.sync_copy(ones_v, acc_c.at[dst_v.at[j]], add=True)
                return c2

            lax.fori_loop(0, CHUNK, body, carry)
            return carry

        lax.fori_loop(0, NB // CHUNK, outer, 0)
        plsc.subcore_barrier()
        pltpu.sync_copy(acc_s.at[sl], osum_hbm.at[c, sl])
        pltpu.sync_copy(acc_c.at[sl], ocnt_hbm.at[c, sl])

    return k(table, srcs, dsts, zfeat, zcnt, ones16)


def _seg_sum64(table, srcs, dsts, z64):
    @functools.partial(
        pl.kernel,
        out_type=jax.ShapeDtypeStruct((NC, NACC, NCLASS), jnp.float32),
        mesh=_MESH,
        compiler_params=pltpu.CompilerParams(use_tc_tiling_on_sc=False),
        scratch_types=[
            pltpu.VMEM((CHUNK, BATCH), jnp.int32),
            pltpu.VMEM((CHUNK, BATCH), jnp.int32),
            pltpu.VMEM((BATCH, NCLASS), jnp.float32),
            pltpu.VMEM_SHARED((NACC, NCLASS), jnp.float32),
        ],
    )
    def k(table_hbm, src_hbm, dst_hbm, z_hbm, osum_hbm,
          src_v, dst_v, rows_v, acc_s):
        c = lax.axis_index("c")
        s = lax.axis_index("s")
        wid = s * NC + c
        r0 = s * ROWS_PER_TILE
        sl = pl.ds(r0, ROWS_PER_TILE)
        pltpu.sync_copy(z_hbm.at[sl], acc_s.at[sl])
        plsc.subcore_barrier()

        def outer(ci, carry):
            pltpu.sync_copy(src_hbm.at[wid, pl.ds(ci * CHUNK, CHUNK)], src_v)
            pltpu.sync_copy(dst_hbm.at[wid, pl.ds(ci * CHUNK, CHUNK)], dst_v)

            def body(j, c2):
                pltpu.sync_copy(table_hbm.at[src_v.at[j]], rows_v)
                pltpu.sync_copy(rows_v, acc_s.at[dst_v.at[j]], add=True)
                return c2

            lax.fori_loop(0, CHUNK, body, carry)
            return carry

        lax.fori_loop(0, NB // CHUNK, outer, 0)
        plsc.subcore_barrier()
        pltpu.sync_copy(acc_s.at[sl], osum_hbm.at[c, sl])

    return k(table, srcs, dsts, z64)


BM = 1000


def _tc1_body(x_ref, s0_ref, s1_ref, c0_ref, c1_ref, w1l_ref, b1_ref,
              w1r_ref, w2l_ref, h_ref, p_ref, cd_ref):
    summed = s0_ref[0] + s1_ref[0]
    cnt = c0_ref[0][:, :1] + c1_ref[0][:, :1]
    cnt = jnp.maximum(cnt, 1.0)
    mean = summed / cnt
    o = (jnp.dot(mean, w1l_ref[...], preferred_element_type=jnp.float32,
                 precision="highest")
         + jnp.dot(x_ref[...], w1r_ref[...], preferred_element_type=jnp.float32,
                   precision="highest")
         + b1_ref[...])
    nrm = jnp.sqrt(jnp.sum(o * o, axis=1, keepdims=True))
    o = o / jnp.maximum(nrm, 1e-12)
    h = jnp.maximum(o, 0.0)
    h_ref[...] = h
    p_ref[...] = jnp.dot(h, w2l_ref[...], preferred_element_type=jnp.float32,
                         precision="highest")
    cd_ref[...] = cnt


def _tc_layer1(x, sums, cnts, W1l, b1, W1r, W2l):
    grid = (N // BM,)
    return pl.pallas_call(
        _tc1_body,
        grid=grid,
        in_specs=[
            pl.BlockSpec((BM, NFEAT), lambda i: (i, 0)),
            pl.BlockSpec((1, BM, NFEAT), lambda i: (0, i, 0)),
            pl.BlockSpec((1, BM, NFEAT), lambda i: (1, i, 0)),
            pl.BlockSpec((1, BM, 8), lambda i: (0, i, 0)),
            pl.BlockSpec((1, BM, 8), lambda i: (1, i, 0)),
            pl.BlockSpec((NFEAT, NHID), lambda i: (0, 0)),
            pl.BlockSpec((1, NHID), lambda i: (0, 0)),
            pl.BlockSpec((NFEAT, NHID), lambda i: (0, 0)),
            pl.BlockSpec((NHID, NCLASS), lambda i: (0, 0)),
        ],
        out_specs=[
            pl.BlockSpec((BM, NHID), lambda i: (i, 0)),
            pl.BlockSpec((BM, NCLASS), lambda i: (i, 0)),
            pl.BlockSpec((BM, 1), lambda i: (i, 0)),
        ],
        out_shape=[
            jax.ShapeDtypeStruct((N, NHID), jnp.float32),
            jax.ShapeDtypeStruct((N, NCLASS), jnp.float32),
            jax.ShapeDtypeStruct((N, 1), jnp.float32),
        ],
    )(x, sums, sums, cnts, cnts, W1l, b1, W1r, W2l)


def _tc2_body(h_ref, s0_ref, s1_ref, cd_ref, w2r_ref, b2_ref, out_ref):
    sp = s0_ref[0] + s1_ref[0]
    mean = sp / cd_ref[...]
    o = (mean
         + jnp.dot(h_ref[...], w2r_ref[...], preferred_element_type=jnp.float32,
                   precision="highest")
         + b2_ref[...])
    nrm = jnp.sqrt(jnp.sum(o * o, axis=1, keepdims=True))
    o = o / jnp.maximum(nrm, 1e-12)
    m = jnp.max(o, axis=1, keepdims=True)
    lse = m + jnp.log(jnp.sum(jnp.exp(o - m), axis=1, keepdims=True))
    out_ref[...] = o - lse


def _tc_layer2(h, sums, cdiv, W2r, b2):
    grid = (N // BM,)
    return pl.pallas_call(
        _tc2_body,
        grid=grid,
        in_specs=[
            pl.BlockSpec((BM, NHID), lambda i: (i, 0)),
            pl.BlockSpec((1, BM, NCLASS), lambda i: (0, i, 0)),
            pl.BlockSpec((1, BM, NCLASS), lambda i: (1, i, 0)),
            pl.BlockSpec((BM, 1), lambda i: (i, 0)),
            pl.BlockSpec((NHID, NCLASS), lambda i: (0, 0)),
            pl.BlockSpec((1, NCLASS), lambda i: (0, 0)),
        ],
        out_specs=pl.BlockSpec((BM, NCLASS), lambda i: (i, 0)),
        out_shape=jax.ShapeDtypeStruct((N, NCLASS), jnp.float32),
    )(h, sums, sums, cdiv, W2r, b2)


def kernel(features, edge_index, W1l, b1, W1r, W2l, b2, W2r):
    src = edge_index[0]
    dst = edge_index[1]
    pad = EPAD - E
    srcp = jnp.concatenate([src, jnp.zeros((pad,), jnp.int32)])
    srcp = srcp.reshape(NW, NB, BATCH)
    dstp = jnp.concatenate([dst, jnp.full((pad,), N, jnp.int32)])
    dstp = dstp.reshape(NW, NB, BATCH)

    zfeat = jnp.zeros((NACC, NFEAT), jnp.float32)
    zcnt = jnp.zeros((NACC, 8), jnp.float32)
    z64 = jnp.zeros((NACC, NCLASS), jnp.float32)
    ones16 = jnp.ones((BATCH, 8), jnp.float32)

    sums1, cnts1 = _seg_sum_cnt(features, srcp, dstp, zfeat, zcnt, ones16)
    h, p, cdiv = _tc_layer1(features, sums1, cnts1, W1l,
                            b1.reshape(1, -1), W1r, W2l)
    sums2 = _seg_sum64(p, srcp, dstp, z64)
    return _tc_layer2(h, sums2, cdiv, W2r, b2.reshape(1, -1))
